# baseline (speedup 1.0000x reference)
import functools

import jax
import jax.numpy as jnp
from jax.experimental import pallas as pl
from jax.experimental.pallas import tpu as pltpu

_B, _N, _C = 16, 5000, 200
_NW = 5
_CW = _C // _NW
_DUR_THRESH = 0.05


_BPS = 4


def _post_kernel(*refs):
    x_refs = refs[:_NW]
    seg_ref, vd_ref, off_ref, scores_ref, labels_ref, segout_ref, mask_ref = refs[_NW:]
    g = pl.program_id(0)
    for j in range(_BPS):
        _one_batch(j, g * _BPS + j, x_refs, seg_ref, vd_ref, off_ref,
                   scores_ref, labels_ref, segout_ref, mask_ref)


def _one_batch(j, b, x_refs, seg_ref, vd_ref, off_ref,
               scores_ref, labels_ref, segout_ref, mask_ref):
    mas = []
    for k in range(_NW):
        x = x_refs[k][j]
        xi = jax.lax.bitcast_convert_type(x, jnp.int32)
        tile = (jax.lax.broadcasted_iota(jnp.int32, (_CW, _N), 0) // 8
                + jnp.int32(k * _CW // 8))
        packed = jax.lax.bitcast_convert_type((xi | jnp.int32(31)) - tile,
                                              jnp.float32)
        xg = packed.reshape(_CW // 8, 8, _N)
        mas.append(jnp.max(xg, axis=0))
    ma = functools.reduce(jnp.maximum, mas)
    mb = jnp.max(ma, axis=0, keepdims=True)
    mi = jax.lax.bitcast_convert_type(mb, jnp.int32)
    t_star = jnp.int32(31) - (mi & jnp.int32(31))
    rev_s = jnp.int32(7) - jax.lax.broadcasted_iota(jnp.int32, (8, _N), 0)
    s_hit = jnp.where(ma == mb, rev_s, jnp.int32(-1))
    s_star = jnp.int32(7) - jnp.max(s_hit, axis=0, keepdims=True)
    labels = t_star * 8 + s_star
    val = jax.lax.bitcast_convert_type((mi & jnp.int32(-32)) | jnp.int32(16),
                                       jnp.float32)
    scores_ref[pl.ds(b, 1), :] = jax.nn.sigmoid(val)
    labels_ref[pl.ds(b, 1), :] = labels

    sr = seg_ref[j]
    c = sr[0:1, :]
    half_w = 0.5 * jnp.exp(sr[1:2, :])
    off = off_ref[b]
    vd = vd_ref[b]
    t1 = jnp.clip(c - half_w + off, 0.0, vd)
    t2 = jnp.clip(c + half_w + off, 0.0, vd)
    segout_ref[j] = jnp.concatenate([t1, t2], axis=0)
    mask_ref[pl.ds(b, 1), :] = t2 - t1 > _DUR_THRESH


def _win_spec(k):
    return pl.BlockSpec((_BPS, _CW, _N), lambda g, k=k: (g, k, 0))


@jax.jit
def _run(logits_t, seg_t, video_durations, offsets):
    out = pl.pallas_call(
        _post_kernel,
        grid=(_B // _BPS,),
        in_specs=[_win_spec(k) for k in range(_NW)] + [
            pl.BlockSpec((_BPS, 2, _N), lambda g: (g, 0, 0)),
            pl.BlockSpec(memory_space=pltpu.SMEM),
            pl.BlockSpec(memory_space=pltpu.SMEM),
        ],
        out_specs=[
            pl.BlockSpec((_B, _N), lambda g: (0, 0)),
            pl.BlockSpec((_B, _N), lambda g: (0, 0)),
            pl.BlockSpec((_BPS, 2, _N), lambda g: (g, 0, 0)),
            pl.BlockSpec((_B, _N), lambda g: (0, 0)),
        ],
        out_shape=[
            jax.ShapeDtypeStruct((_B, _N), jnp.float32),
            jax.ShapeDtypeStruct((_B, _N), jnp.int32),
            jax.ShapeDtypeStruct((_B, 2, _N), jnp.float32),
            jax.ShapeDtypeStruct((_B, _N), jnp.bool_),
        ],
        compiler_params=pltpu.CompilerParams(
            dimension_semantics=("arbitrary",),
        ),
    )(*([logits_t] * _NW), seg_t, video_durations, offsets)
    return out


def kernel(pred_logits, pred_segments, video_durations, feature_durations, offsets):
    logits_t = jnp.transpose(pred_logits, (0, 2, 1))
    seg_t = jnp.transpose(pred_segments, (0, 2, 1))
    scores, labels, seg_out_t, mask = _run(
        logits_t, seg_t, video_durations, offsets)
    segments = jnp.transpose(seg_out_t, (0, 2, 1))
    return scores, labels, segments, mask


# speedup vs baseline: 1.0585x; 1.0585x over previous
import functools

import jax
import jax.numpy as jnp
from jax.experimental import pallas as pl
from jax.experimental.pallas import tpu as pltpu

_B, _N, _C = 16, 5000, 200
_NW = 25
_CW = _C // _NW
_DUR_THRESH = 0.05


_BPS = 2


def _post_kernel(*refs):
    x_refs = refs[:_NW]
    seg_ref, vd_ref, off_ref, scores_ref, labels_ref, segout_ref, mask_ref = refs[_NW:]
    g = pl.program_id(0)
    for j in range(_BPS):
        _one_batch(j, g * _BPS + j, x_refs, seg_ref, vd_ref, off_ref,
                   scores_ref, labels_ref, segout_ref, mask_ref)


def _one_batch(j, b, x_refs, seg_ref, vd_ref, off_ref,
               scores_ref, labels_ref, segout_ref, mask_ref):
    mas = []
    for k in range(_NW):
        x = x_refs[k][j]
        xi = jax.lax.bitcast_convert_type(x, jnp.int32)
        tile = (jax.lax.broadcasted_iota(jnp.int32, (_CW, _N), 0) // 8
                + jnp.int32(k * _CW // 8))
        packed = jax.lax.bitcast_convert_type((xi | jnp.int32(31)) - tile,
                                              jnp.float32)
        xg = packed.reshape(_CW // 8, 8, _N)
        mas.append(jnp.max(xg, axis=0))
    ma = functools.reduce(jnp.maximum, mas)
    mb = jnp.max(ma, axis=0, keepdims=True)
    mi = jax.lax.bitcast_convert_type(mb, jnp.int32)
    t_star = jnp.int32(31) - (mi & jnp.int32(31))
    rev_s = jnp.int32(7) - jax.lax.broadcasted_iota(jnp.int32, (8, _N), 0)
    s_hit = jnp.where(ma == mb, rev_s, jnp.int32(-1))
    s_star = jnp.int32(7) - jnp.max(s_hit, axis=0, keepdims=True)
    labels = t_star * 8 + s_star
    val = jax.lax.bitcast_convert_type((mi & jnp.int32(-32)) | jnp.int32(16),
                                       jnp.float32)
    scores_ref[pl.ds(b, 1), :] = jax.nn.sigmoid(val)
    labels_ref[pl.ds(b, 1), :] = labels

    sr = seg_ref[j]
    c = sr[0:1, :]
    half_w = 0.5 * jnp.exp(sr[1:2, :])
    off = off_ref[b]
    vd = vd_ref[b]
    t1 = jnp.clip(c - half_w + off, 0.0, vd)
    t2 = jnp.clip(c + half_w + off, 0.0, vd)
    segout_ref[j] = jnp.concatenate([t1, t2], axis=0)
    mask_ref[pl.ds(b, 1), :] = t2 - t1 > _DUR_THRESH


def _win_spec(k):
    return pl.BlockSpec((_BPS, _CW, _N), lambda g, k=k: (g, k, 0))


@jax.jit
def _run(logits_t, seg_t, video_durations, offsets):
    out = pl.pallas_call(
        _post_kernel,
        grid=(_B // _BPS,),
        in_specs=[_win_spec(k) for k in range(_NW)] + [
            pl.BlockSpec((_BPS, 2, _N), lambda g: (g, 0, 0)),
            pl.BlockSpec(memory_space=pltpu.SMEM),
            pl.BlockSpec(memory_space=pltpu.SMEM),
        ],
        out_specs=[
            pl.BlockSpec((_B, _N), lambda g: (0, 0)),
            pl.BlockSpec((_B, _N), lambda g: (0, 0)),
            pl.BlockSpec((_BPS, 2, _N), lambda g: (g, 0, 0)),
            pl.BlockSpec((_B, _N), lambda g: (0, 0)),
        ],
        out_shape=[
            jax.ShapeDtypeStruct((_B, _N), jnp.float32),
            jax.ShapeDtypeStruct((_B, _N), jnp.int32),
            jax.ShapeDtypeStruct((_B, 2, _N), jnp.float32),
            jax.ShapeDtypeStruct((_B, _N), jnp.bool_),
        ],
        compiler_params=pltpu.CompilerParams(
            dimension_semantics=("arbitrary",),
        ),
    )(*([logits_t] * _NW), seg_t, video_durations, offsets)
    return out


def kernel(pred_logits, pred_segments, video_durations, feature_durations, offsets):
    logits_t = jnp.transpose(pred_logits, (0, 2, 1))
    seg_t = jnp.transpose(pred_segments, (0, 2, 1))
    scores, labels, seg_out_t, mask = _run(
        logits_t, seg_t, video_durations, offsets)
    segments = jnp.transpose(seg_out_t, (0, 2, 1))
    return scores, labels, segments, mask
